# Initial kernel scaffold; baseline (speedup 1.0000x reference)
import jax
import jax.numpy as jnp
from jax.experimental import pallas as pl


def kernel(coords, edge_index, edge_attr, lookup, params):
    raise NotImplementedError("write your pallas kernel here")


# speedup vs baseline: 6.4116x; 6.4116x over previous
import jax
import jax.numpy as jnp
from jax.experimental import pallas as pl

_INTERPRET = False

_B = 8
_N = 64
_E = 1024
_D = 128
_H = 4
_HD = _H * _D


def _dot(a, b):
    return jax.lax.dot_general(a, b, (((1,), (0,)), ((), ())),
                               preferred_element_type=jnp.float32)


def _dot_nt(a, b):
    return jax.lax.dot_general(a, b, (((1,), (1,)), ((), ())),
                               preferred_element_type=jnp.float32)


def _dotx(a, b):
    return jax.lax.dot_general(a, b, (((1,), (0,)), ((), ())),
                               preferred_element_type=jnp.float32,
                               precision=jax.lax.Precision.HIGHEST)


def _dotx_nt(a, b):
    return jax.lax.dot_general(a, b, (((1,), (1,)), ((), ())),
                               preferred_element_type=jnp.float32,
                               precision=jax.lax.Precision.HIGHEST)


def _gat_kernel(coords_ref, ei_ref, src_ref, dst_ref, ea_ref, fill_ref,
                Wl0, Wr0, We0, att0, bias0, skip0,
                Wl1, Wr1, We1, att1, bias1, skip1,
                Wf, fb, out_ref):
    f32 = jnp.float32
    coords_b = coords_ref[0]
    src_col = src_ref[0]
    dst_col = dst_ref[0]
    dst_row = ei_ref[0, 1:2, :]
    ea_col = ea_ref[0]
    fill = fill_ref[...]

    iota_e_n = jax.lax.broadcasted_iota(jnp.int32, (_E, _N), 1)
    ohs_b = iota_e_n == src_col
    ohd_b = iota_e_n == dst_col
    ohs = ohs_b.astype(f32)
    ohd = ohd_b.astype(f32)
    iota_n_e = jax.lax.broadcasted_iota(jnp.int32, (_N, _E), 0)
    ohd_row = (iota_n_e == dst_row).astype(f32)
    ii = jax.lax.broadcasted_iota(jnp.int32, (_N, _N), 0)
    jj = jax.lax.broadcasted_iota(jnp.int32, (_N, _N), 1)
    eye = (ii == jj).astype(f32)

    def layer(Wl, Wr, We, att, bias, skip):
        xl_n = _dot(coords_b, Wl[...])
        xr_n = _dot(coords_b, Wr[...])
        xls = _dotx(ohs, xl_n)
        xrd = _dotx(ohd, xr_n)
        e_emb = ea_col * We[...]
        e_self = fill * We[...]
        ze = (xls + xrd) + e_emb
        zs = (xl_n + xr_n) + e_self
        me = jnp.where(ze >= 0, ze, 0.2 * ze)
        ms = jnp.where(zs >= 0, zs, 0.2 * zs)
        heads = []
        for h in range(_H):
            sl = slice(h * _D, (h + 1) * _D)
            att_h = att[0:1, sl]
            ae = jnp.sum(me[:, sl] * att_h, axis=1, keepdims=True)
            a_s = jnp.sum(ms[:, sl] * att_h, axis=1, keepdims=True)
            masked = jnp.where(ohd_b, ae, -jnp.inf)
            emax_row = jnp.max(masked, axis=0, keepdims=True)
            emax_col = _dotx_nt(eye, emax_row)
            amax = jnp.maximum(emax_col, a_s)
            amax_dst = _dotx(ohd, amax)
            ex_e = jnp.exp(ae - amax_dst)
            ex_s = jnp.exp(a_s - amax)
            denom = _dotx(ohd_row, ex_e) + ex_s
            denom_dst = _dotx(ohd, denom)
            a_e = ex_e / (denom_dst + 1e-16)
            a_n = ex_s / (denom + 1e-16)
            heads.append(_dotx(ohd_row, a_e * xls[:, sl]) + a_n * xl_n[:, sl])
        h0, h1, h2, h3 = heads
        om = ((h0 + h2) + (h1 + h3)) / 4 + bias[...]
        return jnp.maximum(om, 0.0) + _dot(coords_b, skip[...])

    xa = layer(Wl0, Wr0, We0, att0, bias0, skip0)
    xb = layer(Wl1, Wr1, We1, att1, bias1, skip1)
    xcat = jnp.concatenate([xa, xb], axis=1)
    xf = jnp.maximum(_dot(xcat, Wf[...]) + fb[...], 0.0)
    out_ref[0] = xf


def _dec_kernel(ne_ref, attn_W, attn_b, W_ih, W_hh, b_ih, b_hh, pen,
                tours_ref, logps_ref):
    f32 = jnp.float32
    ne = [ne_ref[b * _N:(b + 1) * _N, :] for b in range(_B)]
    keys = [_dot(ne[b], attn_W[...]) + attn_b[...] for b in range(_B)]
    inp0 = jnp.concatenate(
        [jnp.sum(ne[b], axis=0, keepdims=True) / _N for b in range(_B)],
        axis=0)
    iota_bn = jax.lax.broadcasted_iota(jnp.int32, (_B, _N), 1)
    iota_bt = jax.lax.broadcasted_iota(jnp.int32, (_B, _N + 1), 1)
    sqrt_d = jnp.float32(_D ** 0.5)

    def body(step, carry):
        h, c, inp, visited, tours_acc, logps_acc, first = carry
        g = ((_dot(inp, W_ih[...]) + b_ih[...])
             + _dot(h, W_hh[...])) + b_hh[...]
        gi = jax.nn.sigmoid(g[:, 0:_D])
        gf = jax.nn.sigmoid(g[:, _D:2 * _D])
        gg = jnp.tanh(g[:, 2 * _D:3 * _D])
        go = jax.nn.sigmoid(g[:, 3 * _D:4 * _D])
        c2 = gf * c + gi * gg
        h2 = go * jnp.tanh(c2)
        scores = jnp.concatenate(
            [_dot_nt(h2[b:b + 1, :], keys[b]) for b in range(_B)],
            axis=0) / sqrt_d
        scores = jnp.where(visited > 0, pen[...], scores)
        m = jnp.max(scores, axis=1, keepdims=True)
        e = jnp.exp(scores - m)
        probs = e / jnp.sum(e, axis=1, keepdims=True)
        mb = jnp.max(probs, axis=1, keepdims=True)
        is_max = probs == mb
        curr = jnp.min(jnp.where(is_max, iota_bn, _N), axis=1,
                       keepdims=True)
        oh_curr = (iota_bn == curr).astype(f32)
        lp = jnp.log(jnp.sum(jnp.where(iota_bn == curr, probs, 0.0),
                             axis=1, keepdims=True) + 1e-10)
        first = jnp.where(step == 0, curr, first)
        sel = iota_bt == step
        tours_acc = jnp.where(sel, curr, tours_acc)
        logps_acc = jnp.where(sel, lp, logps_acc)
        visited = jnp.maximum(visited, oh_curr)
        inp2 = jnp.concatenate(
            [_dotx(oh_curr[b:b + 1, :], ne[b]) for b in range(_B)], axis=0)
        return h2, c2, inp2, visited, tours_acc, logps_acc, first

    init = (jnp.zeros((_B, _D), f32), jnp.zeros((_B, _D), f32), inp0,
            jnp.zeros((_B, _N), f32),
            jnp.zeros((_B, _N + 1), jnp.int32),
            jnp.zeros((_B, _N + 1), f32),
            jnp.zeros((_B, 1), jnp.int32))
    h, c, inp, visited, tours_acc, logps_acc, first = jax.lax.fori_loop(
        0, _N, body, init)
    tours_ref[...] = jnp.where(iota_bt == _N, first, tours_acc)
    logps_ref[...] = logps_acc


def _node_emb(coords, edge_index, edge_attr, params):
    f32 = jnp.float32
    coords = coords.astype(f32)
    ei = edge_index.astype(jnp.int32)
    src = ei[:, 0, :].reshape(_B, _E, 1)
    dst = ei[:, 1, :].reshape(_B, _E, 1)
    ea = edge_attr.astype(f32)
    fill = jnp.mean(edge_attr.reshape(_B * _E, 1).astype(f32),
                    axis=0, keepdims=True)

    def packs(p):
        Wl = p['W_l'].astype(f32)
        Wr = p['W_r'].astype(f32)
        We = p['W_e'].astype(f32).reshape(1, _HD)
        att = p['att'].astype(f32).reshape(1, _HD)
        bias = p['bias'].astype(f32).reshape(1, _D)
        return Wl, Wr, We, att, bias

    Wl0, Wr0, We0, att0, bias0 = packs(params['gat0'])
    Wl1, Wr1, We1, att1, bias1 = packs(params['gat1'])
    skip0 = params['skip0'].astype(f32)
    skip1 = params['skip1'].astype(f32)
    Wf = params['fusion_W'].astype(f32)
    fb = params['fusion_b'].astype(f32).reshape(1, _D)

    full = lambda shp: pl.BlockSpec(shp, lambda b, _n=0: (0,) * len(shp))
    batched = lambda shp: pl.BlockSpec(
        (1,) + shp, lambda b, _n=len(shp): (b,) + (0,) * _n)

    node_emb = pl.pallas_call(
        _gat_kernel,
        grid=(_B,),
        in_specs=[
            batched((_N, 2)), batched((2, _E)), batched((_E, 1)),
            batched((_E, 1)), batched((_E, 1)), full((1, 1)),
            full((2, _HD)), full((2, _HD)), full((1, _HD)),
            full((1, _HD)), full((1, _D)), full((2, _D)),
            full((2, _HD)), full((2, _HD)), full((1, _HD)),
            full((1, _HD)), full((1, _D)), full((2, _D)),
            full((2 * _D, _D)), full((1, _D)),
        ],
        out_specs=pl.BlockSpec((1, _N, _D), lambda b: (b, 0, 0)),
        out_shape=jax.ShapeDtypeStruct((_B, _N, _D), f32),
        interpret=_INTERPRET,
    )(coords, ei, src, dst, ea, fill,
      Wl0, Wr0, We0, att0, bias0, skip0,
      Wl1, Wr1, We1, att1, bias1, skip1,
      Wf, fb)
    return node_emb


def _decode(ne2d, params):
    f32 = jnp.float32
    attn_W = params['attn_W'].astype(f32)
    attn_b = params['attn_b'].astype(f32).reshape(1, _D)
    W_ih = params['W_ih'].astype(f32)
    W_hh = params['W_hh'].astype(f32)
    b_ih = params['b_ih'].astype(f32).reshape(1, 4 * _D)
    b_hh = params['b_hh'].astype(f32).reshape(1, 4 * _D)
    pen = params['revisit_penalty'].astype(f32).reshape(1, 1)

    tours, logps = pl.pallas_call(
        _dec_kernel,
        in_specs=[pl.BlockSpec((_B * _N, _D), lambda: (0, 0)),
                  pl.BlockSpec((_D, _D), lambda: (0, 0)),
                  pl.BlockSpec((1, _D), lambda: (0, 0)),
                  pl.BlockSpec((_D, 4 * _D), lambda: (0, 0)),
                  pl.BlockSpec((_D, 4 * _D), lambda: (0, 0)),
                  pl.BlockSpec((1, 4 * _D), lambda: (0, 0)),
                  pl.BlockSpec((1, 4 * _D), lambda: (0, 0)),
                  pl.BlockSpec((1, 1), lambda: (0, 0))],
        out_specs=[pl.BlockSpec((_B, _N + 1), lambda: (0, 0)),
                   pl.BlockSpec((_B, _N + 1), lambda: (0, 0))],
        out_shape=[jax.ShapeDtypeStruct((_B, _N + 1), jnp.int32),
                   jax.ShapeDtypeStruct((_B, _N + 1), f32)],
        interpret=_INTERPRET,
    )(ne2d, attn_W, attn_b, W_ih, W_hh, b_ih, b_hh, pen)
    return tours, logps


@jax.jit
def kernel(coords, edge_index, edge_attr, lookup, params):
    del lookup
    node_emb = _node_emb(coords, edge_index, edge_attr, params)
    return _decode(node_emb.reshape(_B * _N, _D), params)


# speedup vs baseline: 12.2983x; 1.9181x over previous
import jax
import jax.numpy as jnp
from jax.experimental import pallas as pl

_B = 8
_N = 64
_E = 1024
_D = 128
_H = 4
_HD = _H * _D


def _dot(a, b):
    return jax.lax.dot_general(a, b, (((1,), (0,)), ((), ())),
                               preferred_element_type=jnp.float32)


def _dot_nt(a, b):
    return jax.lax.dot_general(a, b, (((1,), (1,)), ((), ())),
                               preferred_element_type=jnp.float32)


def _dotx(a, b):
    return jax.lax.dot_general(a, b, (((1,), (0,)), ((), ())),
                               preferred_element_type=jnp.float32,
                               precision=jax.lax.Precision.HIGHEST)


def _dotx_nt(a, b):
    return jax.lax.dot_general(a, b, (((1,), (1,)), ((), ())),
                               preferred_element_type=jnp.float32,
                               precision=jax.lax.Precision.HIGHEST)


def _gat_kernel(coords_ref, ei_ref, src_ref, dst_ref, ea_ref, fill_ref,
                Wl0, Wr0, We0, att0, bias0, skip0,
                Wl1, Wr1, We1, att1, bias1, skip1,
                Wf, fb, out_ref):
    f32 = jnp.float32
    coords_b = coords_ref[0]
    src_col = src_ref[0]
    dst_col = dst_ref[0]
    dst_row = ei_ref[0, 1:2, :]
    ea_col = ea_ref[0]
    fill = fill_ref[...]

    iota_e_n = jax.lax.broadcasted_iota(jnp.int32, (_E, _N), 1)
    ohs_b = iota_e_n == src_col
    ohd_b = iota_e_n == dst_col
    ohs = ohs_b.astype(f32)
    ohd = ohd_b.astype(f32)
    iota_n_e = jax.lax.broadcasted_iota(jnp.int32, (_N, _E), 0)
    ohd_row = (iota_n_e == dst_row).astype(f32)
    ii = jax.lax.broadcasted_iota(jnp.int32, (_N, _N), 0)
    jj = jax.lax.broadcasted_iota(jnp.int32, (_N, _N), 1)
    eye = (ii == jj).astype(f32)
    cs_g = _dotx(ohs, coords_b)
    cd_g = _dotx(ohd, coords_b)

    def layer(Wl, Wr, We, att, bias, skip):
        xl_n = _dot(coords_b, Wl[...])
        xr_n = _dot(coords_b, Wr[...])
        bf = jnp.bfloat16
        cs_r = cs_g.astype(bf).astype(f32)
        cd_r = cd_g.astype(bf).astype(f32)
        Wl_r = Wl[...].astype(bf).astype(f32)
        Wr_r = Wr[...].astype(bf).astype(f32)
        xls = cs_r[:, 0:1] * Wl_r[0:1, :] + cs_r[:, 1:2] * Wl_r[1:2, :]
        xrd = cd_r[:, 0:1] * Wr_r[0:1, :] + cd_r[:, 1:2] * Wr_r[1:2, :]
        e_emb = ea_col * We[...]
        e_self = fill * We[...]
        ze = (xls + xrd) + e_emb
        zs = (xl_n + xr_n) + e_self
        me = jnp.where(ze >= 0, ze, 0.2 * ze)
        ms = jnp.where(zs >= 0, zs, 0.2 * zs)
        ae_cols, as_cols, emax_rows = [], [], []
        for h in range(_H):
            sl = slice(h * _D, (h + 1) * _D)
            att_h = att[0:1, sl]
            ae = jnp.sum(me[:, sl] * att_h, axis=1, keepdims=True)
            a_s = jnp.sum(ms[:, sl] * att_h, axis=1, keepdims=True)
            masked = jnp.where(ohd_b, ae, -jnp.inf)
            emax_rows.append(jnp.max(masked, axis=0, keepdims=True))
            ae_cols.append(ae)
            as_cols.append(a_s)
        ae_all = jnp.concatenate(ae_cols, axis=1)
        as_all = jnp.concatenate(as_cols, axis=1)
        emax_r = jnp.concatenate(emax_rows, axis=0)
        emax_c = _dotx_nt(eye, emax_r)
        amax = jnp.maximum(emax_c, as_all)
        amax_dst = _dotx(ohd, amax)
        ex_e = jnp.exp(ae_all - amax_dst)
        ex_s = jnp.exp(as_all - amax)
        denom = _dotx(ohd_row, ex_e) + ex_s
        denom_dst = _dotx(ohd, denom)
        a_e = ex_e / (denom_dst + 1e-16)
        a_n = ex_s / (denom + 1e-16)
        a_exp = jnp.concatenate(
            [jnp.broadcast_to(a_e[:, h:h + 1], (_E, _D)) for h in range(_H)],
            axis=1)
        n_exp = jnp.concatenate(
            [jnp.broadcast_to(a_n[:, h:h + 1], (_N, _D)) for h in range(_H)],
            axis=1)
        o_all = _dotx(ohd_row, a_exp * xls) + n_exp * xl_n
        h0 = o_all[:, 0:_D]
        h1 = o_all[:, _D:2 * _D]
        h2 = o_all[:, 2 * _D:3 * _D]
        h3 = o_all[:, 3 * _D:4 * _D]
        om = ((h0 + h2) + (h1 + h3)) / 4 + bias[...]
        return jnp.maximum(om, 0.0) + _dot(coords_b, skip[...])

    xa = layer(Wl0, Wr0, We0, att0, bias0, skip0)
    xb = layer(Wl1, Wr1, We1, att1, bias1, skip1)
    xcat = jnp.concatenate([xa, xb], axis=1)
    xf = jnp.maximum(_dot(xcat, Wf[...]) + fb[...], 0.0)
    out_ref[0] = xf


def _dec_kernel(ne_ref, attn_W, attn_b, W_ih, W_hh, b_ih, b_hh, pen,
                tours_ref, logps_ref):
    f32 = jnp.float32
    ne2d = ne_ref[...]
    keys2d = _dot(ne2d, attn_W[...]) + attn_b[...]
    inp0 = jnp.concatenate(
        [jnp.sum(ne_ref[b * _N:(b + 1) * _N, :], axis=0, keepdims=True) / _N
         for b in range(_B)], axis=0)
    iota_bn = jax.lax.broadcasted_iota(jnp.int32, (_B, _N), 1)
    iota_bt = jax.lax.broadcasted_iota(jnp.int32, (_B, _N + 1), 1)
    di = jax.lax.broadcasted_iota(jnp.int32, (_B, _B, _N), 0)
    dj = jax.lax.broadcasted_iota(jnp.int32, (_B, _B, _N), 1)
    diag = di == dj
    glob_iota = jax.lax.broadcasted_iota(jnp.int32, (_B, _B * _N), 1)
    gbase = jax.lax.broadcasted_iota(jnp.int32, (_B, 1), 0) * _N
    sqrt_d = jnp.float32(_D ** 0.5)

    def body(step, carry):
        h, c, inp, visited, tours_acc, logps_acc, first = carry
        g = ((_dot(inp, W_ih[...]) + b_ih[...])
             + _dot(h, W_hh[...])) + b_hh[...]
        gi = jax.nn.sigmoid(g[:, 0:_D])
        gf = jax.nn.sigmoid(g[:, _D:2 * _D])
        gg = jnp.tanh(g[:, 2 * _D:3 * _D])
        go = jax.nn.sigmoid(g[:, 3 * _D:4 * _D])
        c2 = gf * c + gi * gg
        h2 = go * jnp.tanh(c2)
        full = _dot_nt(h2, keys2d).reshape(_B, _B, _N)
        scores = jnp.sum(jnp.where(diag, full, 0.0),
                         axis=1) / sqrt_d
        scores = jnp.where(visited > 0, pen[...], scores)
        m = jnp.max(scores, axis=1, keepdims=True)
        e = jnp.exp(scores - m)
        probs = e / jnp.sum(e, axis=1, keepdims=True)
        mb = jnp.max(probs, axis=1, keepdims=True)
        is_max = probs == mb
        curr = jnp.min(jnp.where(is_max, iota_bn, _N), axis=1,
                       keepdims=True)
        oh_curr = (iota_bn == curr).astype(f32)
        lp = jnp.log(jnp.sum(jnp.where(iota_bn == curr, probs, 0.0),
                             axis=1, keepdims=True) + 1e-10)
        first = jnp.where(step == 0, curr, first)
        sel = iota_bt == step
        tours_acc = jnp.where(sel, curr, tours_acc)
        logps_acc = jnp.where(sel, lp, logps_acc)
        visited = jnp.maximum(visited, oh_curr)
        oh_glob = (glob_iota == gbase + curr).astype(f32)
        inp2 = _dotx(oh_glob, ne2d)
        return h2, c2, inp2, visited, tours_acc, logps_acc, first

    init = (jnp.zeros((_B, _D), f32), jnp.zeros((_B, _D), f32), inp0,
            jnp.zeros((_B, _N), f32),
            jnp.zeros((_B, _N + 1), jnp.int32),
            jnp.zeros((_B, _N + 1), f32),
            jnp.zeros((_B, 1), jnp.int32))
    h, c, inp, visited, tours_acc, logps_acc, first = jax.lax.fori_loop(
        0, _N, body, init)
    tours_ref[...] = jnp.where(iota_bt == _N, first, tours_acc)
    logps_ref[...] = logps_acc


def _node_emb(coords, edge_index, edge_attr, params):
    f32 = jnp.float32
    coords = coords.astype(f32)
    ei = edge_index.astype(jnp.int32)
    src = ei[:, 0, :].reshape(_B, _E, 1)
    dst = ei[:, 1, :].reshape(_B, _E, 1)
    ea = edge_attr.astype(f32)
    fill = jnp.mean(edge_attr.reshape(_B * _E, 1).astype(f32),
                    axis=0, keepdims=True)

    def packs(p):
        Wl = p['W_l'].astype(f32)
        Wr = p['W_r'].astype(f32)
        We = p['W_e'].astype(f32).reshape(1, _HD)
        att = p['att'].astype(f32).reshape(1, _HD)
        bias = p['bias'].astype(f32).reshape(1, _D)
        return Wl, Wr, We, att, bias

    Wl0, Wr0, We0, att0, bias0 = packs(params['gat0'])
    Wl1, Wr1, We1, att1, bias1 = packs(params['gat1'])
    skip0 = params['skip0'].astype(f32)
    skip1 = params['skip1'].astype(f32)
    Wf = params['fusion_W'].astype(f32)
    fb = params['fusion_b'].astype(f32).reshape(1, _D)

    full = lambda shp: pl.BlockSpec(shp, lambda b, _n=0: (0,) * len(shp))
    batched = lambda shp: pl.BlockSpec(
        (1,) + shp, lambda b, _n=len(shp): (b,) + (0,) * _n)

    node_emb = pl.pallas_call(
        _gat_kernel,
        grid=(_B,),
        in_specs=[
            batched((_N, 2)), batched((2, _E)), batched((_E, 1)),
            batched((_E, 1)), batched((_E, 1)), full((1, 1)),
            full((2, _HD)), full((2, _HD)), full((1, _HD)),
            full((1, _HD)), full((1, _D)), full((2, _D)),
            full((2, _HD)), full((2, _HD)), full((1, _HD)),
            full((1, _HD)), full((1, _D)), full((2, _D)),
            full((2 * _D, _D)), full((1, _D)),
        ],
        out_specs=pl.BlockSpec((1, _N, _D), lambda b: (b, 0, 0)),
        out_shape=jax.ShapeDtypeStruct((_B, _N, _D), f32),
    )(coords, ei, src, dst, ea, fill,
      Wl0, Wr0, We0, att0, bias0, skip0,
      Wl1, Wr1, We1, att1, bias1, skip1,
      Wf, fb)
    return node_emb


def _decode(ne2d, params):
    f32 = jnp.float32
    attn_W = params['attn_W'].astype(f32)
    attn_b = params['attn_b'].astype(f32).reshape(1, _D)
    W_ih = params['W_ih'].astype(f32)
    W_hh = params['W_hh'].astype(f32)
    b_ih = params['b_ih'].astype(f32).reshape(1, 4 * _D)
    b_hh = params['b_hh'].astype(f32).reshape(1, 4 * _D)
    pen = params['revisit_penalty'].astype(f32).reshape(1, 1)

    tours, logps = pl.pallas_call(
        _dec_kernel,
        in_specs=[pl.BlockSpec((_B * _N, _D), lambda: (0, 0)),
                  pl.BlockSpec((_D, _D), lambda: (0, 0)),
                  pl.BlockSpec((1, _D), lambda: (0, 0)),
                  pl.BlockSpec((_D, 4 * _D), lambda: (0, 0)),
                  pl.BlockSpec((_D, 4 * _D), lambda: (0, 0)),
                  pl.BlockSpec((1, 4 * _D), lambda: (0, 0)),
                  pl.BlockSpec((1, 4 * _D), lambda: (0, 0)),
                  pl.BlockSpec((1, 1), lambda: (0, 0))],
        out_specs=[pl.BlockSpec((_B, _N + 1), lambda: (0, 0)),
                   pl.BlockSpec((_B, _N + 1), lambda: (0, 0))],
        out_shape=[jax.ShapeDtypeStruct((_B, _N + 1), jnp.int32),
                   jax.ShapeDtypeStruct((_B, _N + 1), f32)],
    )(ne2d, attn_W, attn_b, W_ih, W_hh, b_ih, b_hh, pen)
    return tours, logps


@jax.jit
def kernel(coords, edge_index, edge_attr, lookup, params):
    del lookup
    node_emb = _node_emb(coords, edge_index, edge_attr, params)
    return _decode(node_emb.reshape(_B * _N, _D), params)
